# baseline (speedup 1.0000x reference)
import math

import jax
import jax.numpy as jnp
from jax.experimental import pallas as pl

_D = 256
_H = 8
_HD = 32
_N = 1024
_NE = 1024
_AR = 8
_NT = 32
_ET = 16


def _gelu(x):
  return 0.5 * x * (1.0 + jax.lax.erf(x * (1.0 / math.sqrt(2.0))))


def _ln(x, g, b):
  m = jnp.mean(x, axis=-1, keepdims=True)
  v = jnp.mean((x - m) ** 2, axis=-1, keepdims=True)
  return (x - m) * jax.lax.rsqrt(v + 1e-5) * g + b


def _mm(a, b):
  return jnp.dot(a.astype(jnp.bfloat16), b.astype(jnp.bfloat16),
                 preferred_element_type=jnp.float32)


def _dgb(a, b, dims):
  return jax.lax.dot_general(a.astype(jnp.bfloat16), b.astype(jnp.bfloat16),
                             dims, preferred_element_type=jnp.float32)


def _rel(x, w):
  h = _mm(x, w['node_proj_w']) + w['node_proj_b']
  q = _mm(h, w['query_w']) + w['query_b']
  k = _mm(h, w['key_w']) + w['key_b']
  v = _mm(h, w['value_w']) + w['value_b']
  outs = []
  for hh in range(_H):
    sl = slice(hh * _HD, (hh + 1) * _HD)
    s = jax.lax.dot_general(q[:, sl].astype(jnp.bfloat16),
                            k[:, sl].astype(jnp.bfloat16),
                            (((1,), (1,)), ((), ())),
                            preferred_element_type=jnp.float32)
    s = s * (1.0 / math.sqrt(_HD))
    m = jnp.max(s, axis=-1, keepdims=True)
    e = jnp.exp(s - m)
    p = e / jnp.sum(e, axis=-1, keepdims=True)
    outs.append(jax.lax.dot_general(p.astype(jnp.bfloat16),
                                    v[:, sl].astype(jnp.bfloat16),
                                    (((1,), (0,)), ((), ())),
                                    preferred_element_type=jnp.float32))
  attn_out = jnp.concatenate(outs, axis=-1)
  o = _mm(attn_out, w['out_proj_w']) + w['out_proj_b'] + x
  o = _ln(o, w['norm_g'], w['norm_b'])
  ff = _mm(_gelu(_mm(o, w['ff1_w']) + w['ff1_b']), w['ff2_w']) + w['ff2_b']
  return _ln(ff + o, w['ffn_g'], w['ffn_b'])


def _hyp(x, w_inc, cnt, counts, edge_emb, hw):
  pooled = _dgb(w_inc, x, (((1,), (0,)), ((), ()))) / cnt
  ef = _mm(pooled, hw['enc_w1']) + _mm(edge_emb, hw['enc_w2']) + hw['enc_b']
  ef = _ln(_gelu(ef), hw['enc_g'], hw['enc_bb'])
  nup = _dgb(w_inc, ef, (((0,), (0,)), ((), ()))) / counts
  u = _mm(x, hw['upd_w1']) + _mm(nup, hw['upd_w2']) + hw['upd_b']
  return _ln(_gelu(u), hw['upd_g'], hw['upd_bb'])


_REL_KEYS = ['node_proj_w', 'node_proj_b', 'query_w', 'query_b',
             'key_w', 'key_b', 'value_w', 'value_b',
             'out_proj_w', 'out_proj_b', 'norm_g', 'norm_b',
             'ff1_w', 'ff1_b', 'ff2_w', 'ff2_b', 'ffn_g', 'ffn_b']
_HYP_KEYS = ['he_etype', 'enc_w1', 'enc_w2', 'enc_b', 'enc_g', 'enc_bb',
             'upd_w1', 'upd_w2', 'upd_b', 'upd_g', 'upd_bb']
_L = 2
_NW = len(_REL_KEYS) + len(_HYP_KEYS)


def _read_named(refs, keys):
  return {k: r[...] for k, r in zip(keys, refs)}


def _mega_body(ids_ref, emb_ref, maskf_ref, mem_ref, types_ref, *rest):
  lw = []
  for li in range(_L):
    base = li * _NW
    rel_refs = rest[base:base + len(_REL_KEYS)]
    hyp_refs = rest[base + len(_REL_KEYS):base + _NW]
    lw.append((_read_named(rel_refs, _REL_KEYS),
               _read_named(hyp_refs, _HYP_KEYS)))
  pw_ref, pb_ref, pg_ref, pbb_ref, ox_ref, og_ref = rest[_L * _NW:]

  ids = ids_ref[0]
  tid = jax.lax.broadcasted_iota(jnp.int32, (_NT, _N), 0)
  oh = jnp.where(ids == tid, 1.0, 0.0)
  x = _dgb(oh, emb_ref[...], (((0,), (0,)), ((), ())))

  maskf = maskf_ref[0]
  mem = mem_ref[0]
  nid = jax.lax.broadcasted_iota(jnp.int32, (_NE, _N), 1)
  w_inc = jnp.zeros((_NE, _N), jnp.float32)
  for a in range(_AR):
    hit = mem[:, a:a + 1] == nid
    w_inc = w_inc + jnp.where(hit, maskf[:, a:a + 1], 0.0)
  cnt = jnp.clip(jnp.sum(maskf, axis=-1, keepdims=True), 1.0)
  counts = _dgb(w_inc, jnp.ones((_NE, 1), jnp.float32),
                (((0,), (0,)), ((), ())))
  counts = jnp.clip(counts, 1.0)

  types = types_ref[0]
  eid = jax.lax.broadcasted_iota(jnp.int32, (_ET, _NE), 0)
  eoh = jnp.where(types == eid, 1.0, 0.0)

  for rw, hw in lw:
    x = _rel(x, rw)
    edge_emb = _dgb(eoh, hw['he_etype'], (((0,), (0,)), ((), ())))
    x = _hyp(x, w_inc, cnt, counts, edge_emb, hw)

  ox_ref[0] = x
  gm = jnp.mean(x, axis=0, keepdims=True)
  gm = _mm(gm, pw_ref[...]) + pb_ref[...]
  og_ref[0] = _ln(_gelu(gm), pg_ref[...], pbb_ref[...])


def _layer_weight_args(p):
  out = []
  for k in _REL_KEYS:
    a = p[k]
    out.append(a.reshape(1, -1) if a.ndim == 1 else a)
  out += [p['he_etype'], p['enc_w'][:_D], p['enc_w'][_D:],
          p['enc_b'].reshape(1, _D), p['enc_g'].reshape(1, _D),
          p['enc_bb'].reshape(1, _D),
          p['upd_w'][:_D], p['upd_w'][_D:], p['upd_b'].reshape(1, _D),
          p['upd_g'].reshape(1, _D), p['upd_bb'].reshape(1, _D)]
  return out


def _full_spec(arr):
  nd = arr.ndim
  return pl.BlockSpec(arr.shape, lambda b, _n=nd: (0,) * _n)


def kernel(node_type_ids, edge_index, edge_types, hyperedge_members,
           hyperedge_types, hyperedge_mask, params):
  del edge_index, edge_types
  B = node_type_ids.shape[0]

  ids = node_type_ids.astype(jnp.int32)
  members = hyperedge_members.astype(jnp.int32)
  types = hyperedge_types.astype(jnp.int32)
  maskf = hyperedge_mask.astype(jnp.float32)

  args = [ids.reshape(B, 1, _N), params['node_type_embed'], maskf, members,
          types.reshape(B, 1, _NE)]
  for p in params['layers']:
    args += _layer_weight_args(p)
  args += [params['pool_w'], params['pool_b'].reshape(1, _D),
           params['pool_g'].reshape(1, _D), params['pool_bb'].reshape(1, _D)]

  in_specs = [
      pl.BlockSpec((1, 1, _N), lambda b: (b, 0, 0)),
      _full_spec(params['node_type_embed']),
      pl.BlockSpec((1, _NE, _AR), lambda b: (b, 0, 0)),
      pl.BlockSpec((1, _NE, _AR), lambda b: (b, 0, 0)),
      pl.BlockSpec((1, 1, _NE), lambda b: (b, 0, 0)),
  ] + [_full_spec(a) for a in args[5:]]

  x_out, graph_emb = pl.pallas_call(
      _mega_body,
      grid=(B,),
      in_specs=in_specs,
      out_specs=[
          pl.BlockSpec((1, _N, _D), lambda b: (b, 0, 0)),
          pl.BlockSpec((1, 1, _D), lambda b: (b, 0, 0)),
      ],
      out_shape=[
          jax.ShapeDtypeStruct((B, _N, _D), jnp.float32),
          jax.ShapeDtypeStruct((B, 1, _D), jnp.float32),
      ],
  )(*args)
  return x_out, graph_emb.reshape(B, _D)
